# Initial kernel scaffold; baseline (speedup 1.0000x reference)
import jax
import jax.numpy as jnp
from jax.experimental import pallas as pl


def kernel(A, D, l, i, concepts, guess_prob):
    raise NotImplementedError("write your pallas kernel here")


# speedup vs baseline: 2.8107x; 2.8107x over previous
import functools

import jax
import jax.numpy as jnp
from jax import lax
from jax.experimental import pallas as pl
from jax.experimental.pallas import tpu as pltpu
from jax.experimental.pallas import tpu_sc as plsc

_N_CORES = 2
_N_SUBCORES = 16
_N_WORKERS = _N_CORES * _N_SUBCORES
_LANES = 16
_IDX_W = 128


def kernel(A, D, l, i, concepts, guess_prob):
    n_students, n_concepts = A.shape

    chunk = -(-n_students // _N_WORKERS)
    chunk = -(-chunk // _IDX_W) * _IDX_W
    n_sub = chunk // _IDX_W

    rc = lax.dynamic_slice_in_dim(concepts, i, 1, axis=1)
    scal_f = jnp.concatenate([
        jnp.asarray(l, jnp.float32).reshape(1),
        jnp.asarray(guess_prob, jnp.float32).reshape(1),
        jnp.zeros((_LANES - 2,), jnp.float32),
    ])
    scal_i = jnp.concatenate([
        rc.reshape(2).astype(jnp.int32),
        jnp.zeros((_LANES - 2,), jnp.int32),
    ])

    A_flat = A.reshape(-1)

    mesh = plsc.VectorSubcoreMesh(
        core_axis_name="c", subcore_axis_name="s",
        num_cores=_N_CORES, num_subcores=_N_SUBCORES)

    @functools.partial(
        pl.kernel,
        out_type=jax.ShapeDtypeStruct((n_students,), jnp.float32),
        mesh=mesh,
        compiler_params=pltpu.CompilerParams(
            use_tc_tiling_on_sc=False, needs_layout_passes=False),
        scratch_types=[
            pltpu.VMEM((_LANES,), jnp.float32),
            pltpu.VMEM((_LANES,), jnp.int32),
            pltpu.VMEM((n_concepts,), jnp.float32),
            pltpu.VMEM((n_sub, _IDX_W), jnp.int32),
            pltpu.VMEM((chunk,), jnp.float32),
            pltpu.SemaphoreType.DMA,
        ],
    )
    def run(A_hbm, D_hbm, sf_hbm, si_hbm, out_hbm, sf_v, si_v, drow_v, idx_v, a_v, sem):
        cid = lax.axis_index("c")
        sid = lax.axis_index("s")
        wid = sid * _N_CORES + cid
        base = jnp.minimum(wid * chunk, n_students - chunk)
        base = pl.multiple_of(base, _IDX_W // 4)

        lane = lax.iota(jnp.int32, _LANES)

        pltpu.sync_copy(sf_hbm, sf_v)
        pltpu.sync_copy(si_hbm, si_v)
        fv = sf_v[...]
        sv = si_v[...]
        ninf = jnp.float32(-jnp.inf)
        lam = jnp.full((_LANES,), jnp.max(jnp.where(lane == 0, fv, ninf)))
        gp = jnp.full((_LANES,), jnp.max(jnp.where(lane == 1, fv, ninf)))
        r = jnp.max(jnp.where(lane == 0, sv, jnp.int32(0)))
        c = jnp.max(jnp.where(lane == 1, sv, jnp.int32(0)))

        pltpu.sync_copy(D_hbm.at[r], drow_v)
        d = plsc.load_gather(drow_v, [jnp.full((_LANES,), c, jnp.int32)])

        for kk in range(n_sub):
            for g in range(_IDX_W // _LANES):
                j0 = kk * _IDX_W + g * _LANES
                rows = base + j0 + lane
                idx_v[kk, pl.ds(g * _LANES, _LANES)] = rows * n_concepts + c

        copies = []
        for kk in range(n_sub):
            cp = pltpu.make_async_copy(
                A_hbm.at[idx_v.at[kk]],
                a_v.at[pl.ds(kk * _IDX_W, _IDX_W)],
                sem,
            )
            cp.start()
            copies.append(cp)
        for cp in copies:
            cp.wait()

        def body(k, carry):
            off = k * _LANES
            a = a_v[pl.ds(off, _LANES)]
            y = jnp.maximum(gp, 1.0 - jnp.exp(-lam * (a - d)))
            a_v[pl.ds(off, _LANES)] = y
            return carry

        lax.fori_loop(0, chunk // _LANES, body, 0)

        pltpu.sync_copy(a_v, out_hbm.at[pl.ds(base, chunk)])

    return run(A_flat, D, scal_f, scal_i)


# speedup vs baseline: 3.3369x; 1.1872x over previous
import functools

import jax
import jax.numpy as jnp
from jax import lax
from jax.experimental import pallas as pl
from jax.experimental.pallas import tpu as pltpu
from jax.experimental.pallas import tpu_sc as plsc

_N_CORES = 2
_N_SUBCORES = 16
_N_WORKERS = _N_CORES * _N_SUBCORES
_LANES = 16
_IDX_W = 128


def kernel(A, D, l, i, concepts, guess_prob):
    n_students, n_concepts = A.shape

    chunk = -(-n_students // _N_WORKERS)
    chunk = -(-chunk // _IDX_W) * _IDX_W
    n_sub = chunk // _IDX_W
    q, rem = divmod(n_sub, 4)
    batches = [q + (1 if b < rem else 0) for b in range(4)]

    rc = lax.dynamic_slice_in_dim(concepts, i, 1, axis=1)
    scal_f = jnp.concatenate([
        jnp.asarray(l, jnp.float32).reshape(1),
        jnp.asarray(guess_prob, jnp.float32).reshape(1),
        rc.reshape(2).astype(jnp.float32),
        jnp.zeros((_LANES - 4,), jnp.float32),
    ])

    A_flat = A.reshape(-1)

    mesh = plsc.VectorSubcoreMesh(
        core_axis_name="c", subcore_axis_name="s",
        num_cores=_N_CORES, num_subcores=_N_SUBCORES)

    @functools.partial(
        pl.kernel,
        out_type=jax.ShapeDtypeStruct((n_students,), jnp.float32),
        mesh=mesh,
        compiler_params=pltpu.CompilerParams(
            use_tc_tiling_on_sc=False, needs_layout_passes=False),
        scratch_types=[
            pltpu.VMEM((_LANES,), jnp.float32),
            pltpu.VMEM((n_concepts,), jnp.float32),
            pltpu.VMEM((n_sub, _IDX_W), jnp.int32),
            pltpu.VMEM((n_sub, _IDX_W), jnp.float32),
            pltpu.VMEM((chunk,), jnp.float32),
            pltpu.SemaphoreType.DMA,
            pltpu.SemaphoreType.DMA,
            pltpu.SemaphoreType.DMA,
            pltpu.SemaphoreType.DMA,
            pltpu.SemaphoreType.DMA,
            pltpu.SemaphoreType.DMA,
            pltpu.SemaphoreType.DMA,
        ],
    )
    def run(A_hbm, D_hbm, sf_hbm, out_hbm, sf_v, drow_v, idx_v, a_v, y_v,
            sem_s, sem_d, sg0, sg1, sg2, sg3, sem_w):
        sgs = [sg0, sg1, sg2, sg3]
        cid = lax.axis_index("c")
        sid = lax.axis_index("s")
        wid = sid * _N_CORES + cid
        base = jnp.minimum(wid * chunk, n_students - chunk)
        base = pl.multiple_of(base, _LANES)

        lane = lax.iota(jnp.int32, _LANES)
        ninf = jnp.float32(-jnp.inf)

        pltpu.async_copy(sf_hbm, sf_v, sem_s).wait()
        fv = sf_v[...]
        lam = jnp.full((_LANES,), jnp.max(jnp.where(lane == 0, fv, ninf)))
        gp = jnp.full((_LANES,), jnp.max(jnp.where(lane == 1, fv, ninf)))
        r = jnp.max(jnp.where(lane == 2, fv, ninf)).astype(jnp.int32)
        c = jnp.max(jnp.where(lane == 3, fv, ninf)).astype(jnp.int32)

        cp_d = pltpu.make_async_copy(D_hbm.at[r], drow_v, sem_d)
        cp_d.start()

        idx0 = (base + lane) * n_concepts + c
        for kk in range(n_sub):
            for g in range(_IDX_W // _LANES):
                j0 = kk * _IDX_W + g * _LANES
                idx_v[kk, pl.ds(g * _LANES, _LANES)] = idx0 + j0 * n_concepts

        copies = []
        row = 0
        for b, nr in enumerate(batches):
            for kk in range(row, row + nr):
                cp = pltpu.make_async_copy(
                    A_hbm.at[idx_v.at[kk]], a_v.at[kk], sgs[b])
                cp.start()
                copies.append((b, kk, cp))
            row += nr

        cp_d.wait()
        d = plsc.load_gather(drow_v, [jnp.full((_LANES,), c, jnp.int32)])
        one = jnp.full((_LANES,), jnp.float32(1.0))
        b_vec = lam * d

        row = 0
        for b, nr in enumerate(batches):
            for _, kk, cp in copies:
                if _ == b:
                    cp.wait()
            for kk in range(row, row + nr):
                for g in range(_IDX_W // _LANES):
                    a = a_v[kk, pl.ds(g * _LANES, _LANES)]
                    y = jnp.maximum(gp, one - jnp.exp(b_vec - lam * a))
                    y_v[pl.ds(kk * _IDX_W + g * _LANES, _LANES)] = y
            off = row * _IDX_W
            pltpu.make_async_copy(
                y_v.at[pl.ds(off, nr * _IDX_W)],
                out_hbm.at[pl.ds(base + off, nr * _IDX_W)],
                sem_w,
            ).start()
            row += nr

        row = 0
        for nr in batches:
            pltpu.make_async_copy(
                y_v.at[pl.ds(row * _IDX_W, nr * _IDX_W)],
                out_hbm.at[pl.ds(base + row * _IDX_W, nr * _IDX_W)],
                sem_w,
            ).wait()
            row += nr

    return run(A_flat, D, scal_f)
